# Initial kernel scaffold; baseline (speedup 1.0000x reference)
import jax
import jax.numpy as jnp
from jax.experimental import pallas as pl


def kernel(x, edge_index, trainflag, W1, a_src1, a_dst1, b1, W2, a_src2, a_dst2, b2):
    raise NotImplementedError("write your pallas kernel here")


# speedup vs baseline: 1.1132x; 1.1132x over previous
import jax
import jax.numpy as jnp
from jax.experimental import pallas as pl
from jax.experimental.pallas import tpu as pltpu


def _mm_body(x_ref, w_ref, asv_ref, adv_ref, h_ref, al_ref):
    h = jnp.dot(x_ref[...], w_ref[...], preferred_element_type=jnp.float32)
    h_ref[...] = h
    al_ref[0, :] = h @ asv_ref[0, :]
    al_ref[1, :] = h @ adv_ref[0, :]


def _mm(x, W, a_src, a_dst):
    n, d_out = x.shape[0], W.shape[1]
    h, al = pl.pallas_call(
        _mm_body,
        out_shape=(
            jax.ShapeDtypeStruct((n, d_out), jnp.float32),
            jax.ShapeDtypeStruct((2, n), jnp.float32),
        ),
    )(x, W, a_src.reshape(1, -1), a_dst.reshape(1, -1))
    return h, al[0], al[1]


def _gat_layer(x, src, dst, W, a_src, a_dst, b, num_nodes):
    h, alpha_src, alpha_dst = _mm(x, W, a_src, a_dst)
    e = alpha_src[src] + alpha_dst[dst]
    e = jax.nn.leaky_relu(e, 0.2)
    m = jax.ops.segment_max(e, dst, num_segments=num_nodes)
    ex = jnp.exp(e - m[dst])
    denom = jax.ops.segment_sum(ex, dst, num_segments=num_nodes)
    att = ex / (denom[dst] + 1e-16)
    out = jax.ops.segment_sum(h[src] * att[:, None], dst, num_segments=num_nodes)
    return out + b


def kernel(x, edge_index, trainflag, W1, a_src1, a_dst1, b1, W2, a_src2, a_dst2, b2):
    num_nodes = x.shape[0]
    ei = edge_index.astype(jnp.int32)
    loop = jnp.arange(num_nodes, dtype=jnp.int32)
    src = jnp.concatenate([ei[0], loop])
    dst = jnp.concatenate([ei[1], loop])
    h = _gat_layer(x, src, dst, W1, a_src1, a_dst1, b1, num_nodes)
    h = jax.nn.leaky_relu(h, 0.01)
    midh = h
    dec = _gat_layer(h, src, dst, W2, a_src2, a_dst2, b2, num_nodes)
    dec = jax.nn.leaky_relu(dec, 0.01)
    return (midh, dec)


# speedup vs baseline: 23.4334x; 21.0509x over previous
import functools

import jax
import jax.numpy as jnp
from jax import lax
from jax.experimental import pallas as pl
from jax.experimental.pallas import tpu as pltpu
from jax.experimental.pallas import tpu_sc as plsc

N = 10000
NP = 10240
NW = 32
BLK1 = 128
NBLK1 = 82
CHUNK = NBLK1 * BLK1
E_PAD = NW * CHUNK
BLK2 = 64
NBLK2 = CHUNK // BLK2
SLICE = NP // 16
D_IN = 128
D_HID = 64

_SC_PARAMS = pltpu.CompilerParams(
    needs_layout_passes=False, use_tc_tiling_on_sc=False)


def _alpha_outs(h, avs, avd):
    a_s = h @ avs
    a_d = h @ avd
    al = jnp.concatenate([a_s[None, :], a_d[None, :]], axis=0)
    am = jnp.concatenate(
        [jnp.full((1, 128), jnp.max(a_s), jnp.float32),
         jnp.full((1, 128), jnp.max(a_d), jnp.float32)], axis=0)
    return al, am


def _mm1_body(x_ref, w_ref, avs_ref, avd_ref, h_ref, al_ref, am_ref):
    h = jnp.dot(x_ref[...], w_ref[...], preferred_element_type=jnp.float32)
    h_ref[...] = h
    al, am = _alpha_outs(h, avs_ref[0, :], avd_ref[0, :])
    al_ref[...] = al
    am_ref[...] = am


def _tc_layer1(xp, W1, a_src1, a_dst1):
    return pl.pallas_call(
        _mm1_body,
        out_shape=(
            jax.ShapeDtypeStruct((NP, D_HID), jnp.float32),
            jax.ShapeDtypeStruct((2, NP), jnp.float32),
            jax.ShapeDtypeStruct((2, 128), jnp.float32),
        ),
    )(xp, W1, a_src1.reshape(1, -1), a_dst1.reshape(1, -1))


def _mm2_body(p_ref, b_ref, w_ref, avs_ref, avd_ref,
              midh_ref, h2_ref, al_ref, am_ref):
    m = p_ref[0] + p_ref[1] + b_ref[0, :]
    m = jnp.maximum(m, 0.01 * m)
    rows = lax.broadcasted_iota(jnp.int32, (NP, D_HID), 0)
    m = jnp.where(rows < N, m, 0.0)
    midh_ref[...] = m
    h2 = jnp.dot(m, w_ref[...], preferred_element_type=jnp.float32)
    h2_ref[...] = h2
    al, am = _alpha_outs(h2, avs_ref[0, :], avd_ref[0, :])
    al_ref[...] = al
    am_ref[...] = am


def _tc_layer2(p, b1, W2, a_src2, a_dst2):
    return pl.pallas_call(
        _mm2_body,
        out_shape=(
            jax.ShapeDtypeStruct((NP, D_HID), jnp.float32),
            jax.ShapeDtypeStruct((NP, D_IN), jnp.float32),
            jax.ShapeDtypeStruct((2, NP), jnp.float32),
            jax.ShapeDtypeStruct((2, 128), jnp.float32),
        ),
    )(p, b1.reshape(1, -1), W2, a_src2.reshape(1, -1), a_dst2.reshape(1, -1))


def _fin_body(p_ref, b_ref, dec_ref):
    d = p_ref[0] + p_ref[1] + b_ref[0, :]
    dec_ref[...] = jnp.maximum(d, 0.01 * d)


def _tc_final(p, b2):
    return pl.pallas_call(
        _fin_body,
        out_shape=jax.ShapeDtypeStruct((NP, D_IN), jnp.float32),
    )(p, b2.reshape(1, -1))


def _make_pass1():
    mesh = plsc.VectorSubcoreMesh(core_axis_name="c", subcore_axis_name="s")

    @functools.partial(
        pl.kernel,
        out_type=(
            jax.ShapeDtypeStruct((2, NP), jnp.float32),
            jax.ShapeDtypeStruct((NW, NBLK1, BLK1), jnp.float32),
        ),
        mesh=mesh,
        compiler_params=_SC_PARAMS,
        scratch_types=[
            pltpu.VMEM((NBLK1, BLK1), jnp.int32),
            pltpu.VMEM((NBLK1, BLK1), jnp.int32),
            pltpu.VMEM((NP,), jnp.float32),
            pltpu.VMEM((NP,), jnp.float32),
            pltpu.VMEM((NBLK1, BLK1), jnp.float32),
            pltpu.VMEM((SLICE,), jnp.float32),
            pltpu.VMEM((2, 128), jnp.float32),
            pltpu.VMEM_SHARED((NP,), jnp.float32),
        ],
    )
    def pass1(src_hbm, dst_hbm, as_hbm, ad_hbm, am_hbm, den_hbm, ex_hbm,
              srcv, dstv, asv, adv, exv, zbuf, amv, shden):
        c = lax.axis_index("c")
        s = lax.axis_index("s")
        wid = s * 2 + c
        pltpu.sync_copy(src_hbm.at[wid], srcv)
        pltpu.sync_copy(dst_hbm.at[wid], dstv)
        pltpu.sync_copy(as_hbm, asv)
        pltpu.sync_copy(ad_hbm, adv)
        pltpu.sync_copy(am_hbm, amv)

        zero16 = jnp.zeros((16,), jnp.float32)

        def zb(i, _):
            zbuf[pl.ds(i * 16, 16)] = zero16
            return 0

        lax.fori_loop(0, SLICE // 16, zb, 0)
        pltpu.sync_copy(zbuf, shden.at[pl.ds(s * SLICE, SLICE)])

        z = amv[0, pl.ds(0, 16)] + amv[1, pl.ds(0, 16)]
        M = jnp.maximum(z, 0.2 * z)

        plsc.subcore_barrier()

        def blk(j, _):
            for l in range(BLK1 // 16):
                sl = pl.ds(l * 16, 16)
                sidx = srcv[j, sl]
                didx = dstv[j, sl]
                a = plsc.load_gather(asv, [sidx])
                b = plsc.load_gather(adv, [didx])
                e = a + b
                e = jnp.maximum(e, 0.2 * e)
                exv[j, sl] = jnp.exp(e - M)
            pltpu.sync_copy(exv.at[j], shden.at[dstv.at[j]], add=True)
            return 0

        lax.fori_loop(0, NBLK1, blk, 0)

        pltpu.sync_copy(exv, ex_hbm.at[wid])
        plsc.subcore_barrier()
        pltpu.sync_copy(shden.at[pl.ds(s * SLICE, SLICE)],
                        den_hbm.at[c, pl.ds(s * SLICE, SLICE)])

    return pass1


def _make_pass2(D, B, NB):
    mesh = plsc.VectorSubcoreMesh(core_axis_name="c", subcore_axis_name="s")

    @functools.partial(
        pl.kernel,
        out_type=jax.ShapeDtypeStruct((2, NP, D), jnp.float32),
        mesh=mesh,
        compiler_params=_SC_PARAMS,
        scratch_types=[
            pltpu.VMEM((2, B), jnp.int32),
            pltpu.VMEM((2, B), jnp.int32),
            pltpu.VMEM((2, B), jnp.float32),
            pltpu.VMEM((B,), jnp.float32),
            pltpu.VMEM((B, D), jnp.float32),
            pltpu.VMEM((B, D), jnp.float32),
            pltpu.VMEM((2, NP), jnp.float32),
            pltpu.VMEM_SHARED((NP, D), jnp.float32),
            pltpu.SemaphoreType.DMA,
            pltpu.SemaphoreType.DMA,
        ],
    )
    def pass2(src_hbm, dst_hbm, ex_hbm, den2_hbm, h_hbm, out_hbm,
              srcb, dstb, exb, attb, rows0, rows1, den2v, shout,
              semi, semg):
        c = lax.axis_index("c")
        s = lax.axis_index("s")
        wid = s * 2 + c
        z16 = jnp.zeros((16,), jnp.int32)

        pltpu.sync_copy(den2_hbm, den2v)

        def dn(i, _):
            sl = pl.ds(i * 16, 16)
            den2v[0, sl] = den2v[0, sl] + den2v[1, sl]
            return 0

        lax.fori_loop(0, NP // 16, dn, 0)

        zero16 = jnp.zeros((16,), jnp.float32)

        def zr(r, _):
            for k in range(D // 16):
                rows0[r, pl.ds(k * 16, 16)] = zero16
            return 0

        lax.fori_loop(0, B, zr, 0)
        for q in range(SLICE // B):
            pltpu.sync_copy(rows0, shout.at[pl.ds(s * SLICE + q * B, B), :])
        plsc.subcore_barrier()

        bufs = (rows0, rows1)

        def idx_start(j, p):
            pltpu.async_copy(src_hbm.at[wid, j], srcb.at[p], semi)
            pltpu.async_copy(dst_hbm.at[wid, j], dstb.at[p], semi)
            pltpu.async_copy(ex_hbm.at[wid, j], exb.at[p], semi)

        def idx_wait(p):
            pltpu.make_async_copy(src_hbm.at[wid, 0], srcb.at[p], semi).wait()
            pltpu.make_async_copy(dst_hbm.at[wid, 0], dstb.at[p], semi).wait()
            pltpu.make_async_copy(ex_hbm.at[wid, 0], exb.at[p], semi).wait()

        def row_start(p):
            pltpu.async_copy(h_hbm.at[srcb.at[p]], bufs[p], semg)

        def row_wait(p):
            pltpu.make_async_copy(h_hbm.at[srcb.at[p]], bufs[p], semg).wait()

        idx_start(0, 0)
        idx_wait(0)
        row_start(0)
        idx_start(1, 1)

        def body(jj, _):
            for p in range(2):
                j = jj * 2 + p
                buf = bufs[p]
                row_wait(p)

                for l in range(B // 16):
                    sl = pl.ds(l * 16, 16)
                    didx = dstb[p, sl]
                    den = plsc.load_gather(den2v, [z16, didx])
                    attb[sl] = exb[p, sl] / (den + 1e-16)

                @pl.when(j + 1 < NB)
                def _():
                    idx_wait(1 - p)
                    row_start(1 - p)

                def scl(r, _):
                    bidx = jnp.broadcast_to(r, (16,)).astype(jnp.int32)
                    sc = plsc.load_gather(attb, [bidx])
                    for k in range(D // 16):
                        sl = pl.ds(k * 16, 16)
                        buf[r, sl] = buf[r, sl] * sc
                    return 0

                lax.fori_loop(0, B, scl, 0)
                pltpu.sync_copy(buf, shout.at[dstb.at[p]], add=True)

                @pl.when(j + 2 < NB)
                def _():
                    idx_start(j + 2, p)
            return 0

        lax.fori_loop(0, NB // 2, body, 0)

        plsc.subcore_barrier()
        pltpu.sync_copy(shout.at[pl.ds(s * SLICE, SLICE), :],
                        out_hbm.at[c, pl.ds(s * SLICE, SLICE), :])

    return pass2


_pass1 = _make_pass1()
_pass2_l1 = _make_pass2(D_HID, BLK1, NBLK1)
_pass2_l2 = _make_pass2(D_IN, BLK2, NBLK2)


def kernel(x, edge_index, trainflag, W1, a_src1, a_dst1, b1, W2, a_src2, a_dst2, b2):
    ei = edge_index.astype(jnp.int32)
    loop = jnp.arange(N, dtype=jnp.int32)
    n_edges = ei.shape[1]
    n_pad = E_PAD - (n_edges + N)
    src = jnp.concatenate([ei[0], loop, jnp.zeros((n_pad,), jnp.int32)])
    dst = jnp.concatenate([ei[1], loop, jnp.full((n_pad,), N, jnp.int32)])
    src1 = src.reshape(NW, NBLK1, BLK1)
    dst1 = dst.reshape(NW, NBLK1, BLK1)
    src2 = src.reshape(NW, NBLK2, BLK2)
    dst2 = dst.reshape(NW, NBLK2, BLK2)
    xp = jnp.pad(x, ((0, NP - N), (0, 0)))

    h1, al1, am1 = _tc_layer1(xp, W1, a_src1, a_dst1)
    den1, ex1 = _pass1(src1, dst1, al1[0], al1[1], am1)
    p1 = _pass2_l1(src1, dst1, ex1, den1, h1)

    midh, h2, al2, am2 = _tc_layer2(p1, b1, W2, a_src2, a_dst2)
    den2, ex2 = _pass1(src1, dst1, al2[0], al2[1], am2)
    p2 = _pass2_l2(src2, dst2, ex2.reshape(NW, NBLK2, BLK2), den2, h2)

    dec = _tc_final(p2, b2)
    return (midh[:N], dec[:N])


# speedup vs baseline: 24.4130x; 1.0418x over previous
import functools

import jax
import jax.numpy as jnp
from jax import lax
from jax.experimental import pallas as pl
from jax.experimental.pallas import tpu as pltpu
from jax.experimental.pallas import tpu_sc as plsc

N = 10000
NP = 10240
NW = 32
BLK1 = 128
NBLK1 = 82
CHUNK = NBLK1 * BLK1
E_PAD = NW * CHUNK
BLK2 = 64
NBLK2 = CHUNK // BLK2
SLICE = NP // 16
D_IN = 128
D_HID = 64

_SC_PARAMS = pltpu.CompilerParams(
    needs_layout_passes=False, use_tc_tiling_on_sc=False)


def _alpha_outs(h, avs, avd):
    a_s = h @ avs
    a_d = h @ avd
    al = jnp.concatenate([a_s[None, :], a_d[None, :]], axis=0)
    am = jnp.concatenate(
        [jnp.full((1, 128), jnp.max(a_s), jnp.float32),
         jnp.full((1, 128), jnp.max(a_d), jnp.float32)], axis=0)
    return al, am


def _mm1_body(x_ref, w_ref, avs_ref, avd_ref, h_ref, al_ref, am_ref):
    h = jnp.dot(x_ref[...], w_ref[...], preferred_element_type=jnp.float32)
    h_ref[...] = h
    al, am = _alpha_outs(h, avs_ref[0, :], avd_ref[0, :])
    al_ref[...] = al
    am_ref[...] = am


def _tc_layer1(xp, W1, a_src1, a_dst1):
    return pl.pallas_call(
        _mm1_body,
        out_shape=(
            jax.ShapeDtypeStruct((NP, D_HID), jnp.float32),
            jax.ShapeDtypeStruct((2, NP), jnp.float32),
            jax.ShapeDtypeStruct((2, 128), jnp.float32),
        ),
    )(xp, W1, a_src1.reshape(1, -1), a_dst1.reshape(1, -1))


def _mm2_body(p_ref, b_ref, w_ref, avs_ref, avd_ref,
              midh_ref, h2_ref, al_ref, am_ref):
    m = p_ref[0] + p_ref[1] + b_ref[0, :]
    m = jnp.maximum(m, 0.01 * m)
    rows = lax.broadcasted_iota(jnp.int32, (NP, D_HID), 0)
    m = jnp.where(rows < N, m, 0.0)
    midh_ref[...] = m
    h2 = jnp.dot(m, w_ref[...], preferred_element_type=jnp.float32)
    h2_ref[...] = h2
    al, am = _alpha_outs(h2, avs_ref[0, :], avd_ref[0, :])
    al_ref[...] = al
    am_ref[...] = am


def _tc_layer2(p, b1, W2, a_src2, a_dst2):
    return pl.pallas_call(
        _mm2_body,
        out_shape=(
            jax.ShapeDtypeStruct((NP, D_HID), jnp.float32),
            jax.ShapeDtypeStruct((NP, D_IN), jnp.float32),
            jax.ShapeDtypeStruct((2, NP), jnp.float32),
            jax.ShapeDtypeStruct((2, 128), jnp.float32),
        ),
    )(p, b1.reshape(1, -1), W2, a_src2.reshape(1, -1), a_dst2.reshape(1, -1))


def _fin_body(p_ref, b_ref, dec_ref):
    d = p_ref[0] + p_ref[1] + b_ref[0, :]
    dec_ref[...] = jnp.maximum(d, 0.01 * d)


def _tc_final(p, b2):
    return pl.pallas_call(
        _fin_body,
        out_shape=jax.ShapeDtypeStruct((NP, D_IN), jnp.float32),
    )(p, b2.reshape(1, -1))


def _make_pass1():
    mesh = plsc.VectorSubcoreMesh(core_axis_name="c", subcore_axis_name="s")

    @functools.partial(
        pl.kernel,
        out_type=(
            jax.ShapeDtypeStruct((2, NP), jnp.float32),
            jax.ShapeDtypeStruct((NW, NBLK1, BLK1), jnp.float32),
        ),
        mesh=mesh,
        compiler_params=_SC_PARAMS,
        scratch_types=[
            pltpu.VMEM((NBLK1, BLK1), jnp.int32),
            pltpu.VMEM((NBLK1, BLK1), jnp.int32),
            pltpu.VMEM((NP,), jnp.float32),
            pltpu.VMEM((NP,), jnp.float32),
            pltpu.VMEM((NBLK1, BLK1), jnp.float32),
            pltpu.VMEM((SLICE,), jnp.float32),
            pltpu.VMEM((2, 128), jnp.float32),
            pltpu.VMEM_SHARED((NP,), jnp.float32),
        ],
    )
    def pass1(src_hbm, dst_hbm, as_hbm, ad_hbm, am_hbm, den_hbm, ex_hbm,
              srcv, dstv, asv, adv, exv, zbuf, amv, shden):
        c = lax.axis_index("c")
        s = lax.axis_index("s")
        wid = s * 2 + c
        pltpu.sync_copy(src_hbm.at[wid], srcv)
        pltpu.sync_copy(dst_hbm.at[wid], dstv)
        pltpu.sync_copy(as_hbm, asv)
        pltpu.sync_copy(ad_hbm, adv)
        pltpu.sync_copy(am_hbm, amv)

        zero16 = jnp.zeros((16,), jnp.float32)

        def zb(i, _):
            zbuf[pl.ds(i * 16, 16)] = zero16
            return 0

        lax.fori_loop(0, SLICE // 16, zb, 0)
        pltpu.sync_copy(zbuf, shden.at[pl.ds(s * SLICE, SLICE)])

        z = amv[0, pl.ds(0, 16)] + amv[1, pl.ds(0, 16)]
        M = jnp.maximum(z, 0.2 * z)

        plsc.subcore_barrier()

        def blk(j, _):
            for l in range(BLK1 // 16):
                sl = pl.ds(l * 16, 16)
                sidx = srcv[j, sl]
                didx = dstv[j, sl]
                a = plsc.load_gather(asv, [sidx])
                b = plsc.load_gather(adv, [didx])
                e = a + b
                e = jnp.maximum(e, 0.2 * e)
                exv[j, sl] = jnp.exp(e - M)
            pltpu.sync_copy(exv.at[j], shden.at[dstv.at[j]], add=True)
            return 0

        lax.fori_loop(0, NBLK1, blk, 0)

        pltpu.sync_copy(exv, ex_hbm.at[wid])
        plsc.subcore_barrier()
        pltpu.sync_copy(shden.at[pl.ds(s * SLICE, SLICE)],
                        den_hbm.at[c, pl.ds(s * SLICE, SLICE)])

    return pass1


def _make_pass2(D, B, NB):
    mesh = plsc.VectorSubcoreMesh(core_axis_name="c", subcore_axis_name="s")

    @functools.partial(
        pl.kernel,
        out_type=jax.ShapeDtypeStruct((2, NP, D), jnp.float32),
        mesh=mesh,
        compiler_params=_SC_PARAMS,
        scratch_types=[
            pltpu.VMEM((2, B), jnp.int32),
            pltpu.VMEM((2, B), jnp.int32),
            pltpu.VMEM((2, B), jnp.float32),
            pltpu.VMEM((2, B), jnp.float32),
            pltpu.VMEM((B,), jnp.float32),
            pltpu.VMEM((B, D), jnp.float32),
            pltpu.VMEM((B, D), jnp.float32),
            pltpu.VMEM((2, SLICE), jnp.float32),
            pltpu.VMEM((SLICE,), jnp.float32),
            pltpu.VMEM_SHARED((NP, D), jnp.float32),
            pltpu.VMEM_SHARED((NP,), jnp.float32),
            pltpu.SemaphoreType.DMA,
            pltpu.SemaphoreType.DMA,
        ],
    )
    def pass2(src_hbm, dst_hbm, ex_hbm, den2_hbm, h_hbm, out_hbm,
              srcb, dstb, exb, denb, attb, rows0, rows1, dbuf, zbuf,
              shout, den_sh, semi, semg):
        c = lax.axis_index("c")
        s = lax.axis_index("s")
        wid = s * 2 + c
        sl_nodes = pl.ds(s * SLICE, SLICE)

        pltpu.sync_copy(den2_hbm.at[:, sl_nodes], dbuf)

        def dn(i, _):
            sl = pl.ds(i * 16, 16)
            zbuf[sl] = dbuf[0, sl] + dbuf[1, sl]
            return 0

        lax.fori_loop(0, SLICE // 16, dn, 0)
        pltpu.sync_copy(zbuf, den_sh.at[sl_nodes])

        zero16 = jnp.zeros((16,), jnp.float32)

        def zr(r, _):
            for k in range(D // 16):
                rows0[r, pl.ds(k * 16, 16)] = zero16
            return 0

        lax.fori_loop(0, B, zr, 0)
        for q in range(SLICE // B):
            pltpu.sync_copy(rows0, shout.at[pl.ds(s * SLICE + q * B, B), :])
        plsc.subcore_barrier()

        bufs = (rows0, rows1)

        def idx_start(j, p):
            pltpu.async_copy(src_hbm.at[wid, j], srcb.at[p], semi)
            pltpu.async_copy(dst_hbm.at[wid, j], dstb.at[p], semi)
            pltpu.async_copy(ex_hbm.at[wid, j], exb.at[p], semi)

        def idx_wait(p):
            pltpu.make_async_copy(src_hbm.at[wid, 0], srcb.at[p], semi).wait()
            pltpu.make_async_copy(dst_hbm.at[wid, 0], dstb.at[p], semi).wait()
            pltpu.make_async_copy(ex_hbm.at[wid, 0], exb.at[p], semi).wait()

        def gath_start(p):
            pltpu.async_copy(h_hbm.at[srcb.at[p]], bufs[p], semg)

        def gath_wait(p):
            pltpu.make_async_copy(h_hbm.at[srcb.at[p]], bufs[p], semg).wait()

        idx_start(0, 0)
        idx_wait(0)
        gath_start(0)
        idx_start(1, 1)

        def body(jj, _):
            for p in range(2):
                j = jj * 2 + p
                buf = bufs[p]
                gath_wait(p)
                pltpu.sync_copy(den_sh.at[dstb.at[p]], denb.at[p])

                for l in range(B // 16):
                    sl = pl.ds(l * 16, 16)
                    attb[sl] = exb[p, sl] / (denb[p, sl] + 1e-16)

                @pl.when(j + 1 < NB)
                def _():
                    idx_wait(1 - p)
                    gath_start(1 - p)

                def scl(r, _):
                    bidx = jnp.broadcast_to(r, (16,)).astype(jnp.int32)
                    sc = plsc.load_gather(attb, [bidx])
                    for k in range(D // 16):
                        sl = pl.ds(k * 16, 16)
                        buf[r, sl] = buf[r, sl] * sc
                    return 0

                lax.fori_loop(0, B, scl, 0)
                pltpu.sync_copy(buf, shout.at[dstb.at[p]], add=True)

                @pl.when(j + 2 < NB)
                def _():
                    idx_start(j + 2, p)
            return 0

        lax.fori_loop(0, NB // 2, body, 0)

        plsc.subcore_barrier()
        pltpu.sync_copy(shout.at[sl_nodes, :],
                        out_hbm.at[c, sl_nodes, :])

    return pass2


_pass1 = _make_pass1()
_pass2_l1 = _make_pass2(D_HID, BLK1, NBLK1)
_pass2_l2 = _make_pass2(D_IN, BLK1, NBLK1)


def kernel(x, edge_index, trainflag, W1, a_src1, a_dst1, b1, W2, a_src2, a_dst2, b2):
    ei = edge_index.astype(jnp.int32)
    loop = jnp.arange(N, dtype=jnp.int32)
    n_edges = ei.shape[1]
    n_pad = E_PAD - (n_edges + N)
    src = jnp.concatenate([ei[0], loop, jnp.zeros((n_pad,), jnp.int32)])
    dst = jnp.concatenate([ei[1], loop, jnp.full((n_pad,), N, jnp.int32)])
    src1 = src.reshape(NW, NBLK1, BLK1)
    dst1 = dst.reshape(NW, NBLK1, BLK1)
    xp = jnp.pad(x, ((0, NP - N), (0, 0)))

    h1, al1, am1 = _tc_layer1(xp, W1, a_src1, a_dst1)
    den1, ex1 = _pass1(src1, dst1, al1[0], al1[1], am1)
    p1 = _pass2_l1(src1, dst1, ex1, den1, h1)

    midh, h2, al2, am2 = _tc_layer2(p1, b1, W2, a_src2, a_dst2)
    den2, ex2 = _pass1(src1, dst1, al2[0], al2[1], am2)
    p2 = _pass2_l2(src1, dst1, ex2, den2, h2)

    dec = _tc_final(p2, b2)
    return (midh[:N], dec[:N])


# speedup vs baseline: 25.2174x; 1.0329x over previous
import functools

import jax
import jax.numpy as jnp
from jax import lax
from jax.experimental import pallas as pl
from jax.experimental.pallas import tpu as pltpu
from jax.experimental.pallas import tpu_sc as plsc

N = 10000
NP = 10240
NW = 32
BLK1 = 128
NBLK1 = 82
CHUNK = NBLK1 * BLK1
E_PAD = NW * CHUNK
BLK2 = 64
NBLK2 = CHUNK // BLK2
SLICE = NP // 16
D_IN = 128
D_HID = 64

_SC_PARAMS = pltpu.CompilerParams(
    needs_layout_passes=False, use_tc_tiling_on_sc=False)


def _alpha_outs(h, avs, avd):
    a_s = h @ avs
    a_d = h @ avd
    al = jnp.concatenate([a_s[None, :], a_d[None, :]], axis=0)
    am = jnp.concatenate(
        [jnp.full((1, 128), jnp.max(a_s), jnp.float32),
         jnp.full((1, 128), jnp.max(a_d), jnp.float32)], axis=0)
    return al, am


def _mm1_body(x_ref, w_ref, avs_ref, avd_ref, h_ref, al_ref, am_ref):
    h = jnp.dot(x_ref[...], w_ref[...], preferred_element_type=jnp.float32)
    h_ref[...] = h
    al, am = _alpha_outs(h, avs_ref[0, :], avd_ref[0, :])
    al_ref[...] = al
    am_ref[...] = am


def _tc_layer1(xp, W1, a_src1, a_dst1):
    return pl.pallas_call(
        _mm1_body,
        out_shape=(
            jax.ShapeDtypeStruct((NP, D_HID), jnp.float32),
            jax.ShapeDtypeStruct((2, NP), jnp.float32),
            jax.ShapeDtypeStruct((2, 128), jnp.float32),
        ),
    )(xp, W1, a_src1.reshape(1, -1), a_dst1.reshape(1, -1))


def _mm2_body(p_ref, den_ref, b_ref, w_ref, avs_ref, avd_ref,
              midh_ref, h2_ref, al_ref, am_ref):
    dsum = den_ref[0] + den_ref[1] + 1e-16
    m = (p_ref[0] + p_ref[1]) / dsum[:, None] + b_ref[0, :]
    m = jnp.maximum(m, 0.01 * m)
    rows = lax.broadcasted_iota(jnp.int32, (NP, D_HID), 0)
    m = jnp.where(rows < N, m, 0.0)
    midh_ref[...] = m
    h2 = jnp.dot(m, w_ref[...], preferred_element_type=jnp.float32)
    h2_ref[...] = h2
    al, am = _alpha_outs(h2, avs_ref[0, :], avd_ref[0, :])
    al_ref[...] = al
    am_ref[...] = am


def _tc_layer2(p, den, b1, W2, a_src2, a_dst2):
    return pl.pallas_call(
        _mm2_body,
        out_shape=(
            jax.ShapeDtypeStruct((NP, D_HID), jnp.float32),
            jax.ShapeDtypeStruct((NP, D_IN), jnp.float32),
            jax.ShapeDtypeStruct((2, NP), jnp.float32),
            jax.ShapeDtypeStruct((2, 128), jnp.float32),
        ),
    )(p, den, b1.reshape(1, -1), W2, a_src2.reshape(1, -1),
      a_dst2.reshape(1, -1))


def _fin_body(p_ref, den_ref, b_ref, dec_ref):
    dsum = den_ref[0] + den_ref[1] + 1e-16
    d = (p_ref[0] + p_ref[1]) / dsum[:, None] + b_ref[0, :]
    dec_ref[...] = jnp.maximum(d, 0.01 * d)


def _tc_final(p, den, b2):
    return pl.pallas_call(
        _fin_body,
        out_shape=jax.ShapeDtypeStruct((NP, D_IN), jnp.float32),
    )(p, den, b2.reshape(1, -1))


def _make_pass1():
    mesh = plsc.VectorSubcoreMesh(core_axis_name="c", subcore_axis_name="s")

    @functools.partial(
        pl.kernel,
        out_type=(
            jax.ShapeDtypeStruct((2, NP), jnp.float32),
            jax.ShapeDtypeStruct((NW, NBLK1, BLK1), jnp.float32),
        ),
        mesh=mesh,
        compiler_params=_SC_PARAMS,
        scratch_types=[
            pltpu.VMEM((NBLK1, BLK1), jnp.int32),
            pltpu.VMEM((NBLK1, BLK1), jnp.int32),
            pltpu.VMEM((NP,), jnp.float32),
            pltpu.VMEM((NP,), jnp.float32),
            pltpu.VMEM((NBLK1, BLK1), jnp.float32),
            pltpu.VMEM((SLICE,), jnp.float32),
            pltpu.VMEM((2, 128), jnp.float32),
            pltpu.VMEM_SHARED((NP,), jnp.float32),
        ],
    )
    def pass1(src_hbm, dst_hbm, as_hbm, ad_hbm, am_hbm, den_hbm, ex_hbm,
              srcv, dstv, asv, adv, exv, zbuf, amv, shden):
        c = lax.axis_index("c")
        s = lax.axis_index("s")
        wid = s * 2 + c
        pltpu.sync_copy(src_hbm.at[wid], srcv)
        pltpu.sync_copy(dst_hbm.at[wid], dstv)
        pltpu.sync_copy(as_hbm, asv)
        pltpu.sync_copy(ad_hbm, adv)
        pltpu.sync_copy(am_hbm, amv)

        zero16 = jnp.zeros((16,), jnp.float32)

        def zb(i, _):
            zbuf[pl.ds(i * 16, 16)] = zero16
            return 0

        lax.fori_loop(0, SLICE // 16, zb, 0)
        pltpu.sync_copy(zbuf, shden.at[pl.ds(s * SLICE, SLICE)])

        z = amv[0, pl.ds(0, 16)] + amv[1, pl.ds(0, 16)]
        M = jnp.maximum(z, 0.2 * z)

        plsc.subcore_barrier()

        def blk(j, _):
            for l in range(BLK1 // 16):
                sl = pl.ds(l * 16, 16)
                sidx = srcv[j, sl]
                didx = dstv[j, sl]
                a = plsc.load_gather(asv, [sidx])
                b = plsc.load_gather(adv, [didx])
                e = a + b
                e = jnp.maximum(e, 0.2 * e)
                exv[j, sl] = jnp.exp(e - M)
            pltpu.sync_copy(exv.at[j], shden.at[dstv.at[j]], add=True)
            return 0

        lax.fori_loop(0, NBLK1, blk, 0)

        pltpu.sync_copy(exv, ex_hbm.at[wid])
        plsc.subcore_barrier()
        pltpu.sync_copy(shden.at[pl.ds(s * SLICE, SLICE)],
                        den_hbm.at[c, pl.ds(s * SLICE, SLICE)])

    return pass1


def _make_pass2(D, B, NB):
    mesh = plsc.VectorSubcoreMesh(core_axis_name="c", subcore_axis_name="s")

    @functools.partial(
        pl.kernel,
        out_type=jax.ShapeDtypeStruct((2, NP, D), jnp.float32),
        mesh=mesh,
        compiler_params=_SC_PARAMS,
        scratch_types=[
            pltpu.VMEM((2, B), jnp.int32),
            pltpu.VMEM((2, B), jnp.int32),
            pltpu.VMEM((2, B), jnp.float32),
            pltpu.VMEM((B, D), jnp.float32),
            pltpu.VMEM((B, D), jnp.float32),
            pltpu.VMEM_SHARED((NP, D), jnp.float32),
            pltpu.SemaphoreType.DMA,
            pltpu.SemaphoreType.DMA,
        ],
    )
    def pass2(src_hbm, dst_hbm, ex_hbm, h_hbm, out_hbm,
              srcb, dstb, exb, rows0, rows1, shout, semi, semg):
        c = lax.axis_index("c")
        s = lax.axis_index("s")
        wid = s * 2 + c
        sl_nodes = pl.ds(s * SLICE, SLICE)

        zero16 = jnp.zeros((16,), jnp.float32)

        def zr(r, _):
            for k in range(D // 16):
                rows0[r, pl.ds(k * 16, 16)] = zero16
            return 0

        lax.fori_loop(0, B, zr, 0)
        for q in range(SLICE // B):
            pltpu.sync_copy(rows0, shout.at[pl.ds(s * SLICE + q * B, B), :])
        plsc.subcore_barrier()

        bufs = (rows0, rows1)

        def idx_start(j, p):
            pltpu.async_copy(src_hbm.at[wid, j], srcb.at[p], semi)
            pltpu.async_copy(dst_hbm.at[wid, j], dstb.at[p], semi)
            pltpu.async_copy(ex_hbm.at[wid, j], exb.at[p], semi)

        def idx_wait(p):
            pltpu.make_async_copy(src_hbm.at[wid, 0], srcb.at[p], semi).wait()
            pltpu.make_async_copy(dst_hbm.at[wid, 0], dstb.at[p], semi).wait()
            pltpu.make_async_copy(ex_hbm.at[wid, 0], exb.at[p], semi).wait()

        def gath_start(p):
            pltpu.async_copy(h_hbm.at[srcb.at[p]], bufs[p], semg)

        def gath_wait(p):
            pltpu.make_async_copy(h_hbm.at[srcb.at[p]], bufs[p], semg).wait()

        idx_start(0, 0)
        idx_wait(0)
        gath_start(0)
        idx_start(1, 1)

        def body(jj, _):
            for p in range(2):
                j = jj * 2 + p
                buf = bufs[p]
                gath_wait(p)

                @pl.when(j + 1 < NB)
                def _():
                    idx_wait(1 - p)
                    gath_start(1 - p)

                pidx = jnp.full((16,), p, jnp.int32)

                def scl(r, _):
                    bidx = jnp.broadcast_to(r, (16,)).astype(jnp.int32)
                    sc = plsc.load_gather(exb, [pidx, bidx])
                    for k in range(D // 16):
                        sl = pl.ds(k * 16, 16)
                        buf[r, sl] = buf[r, sl] * sc
                    return 0

                lax.fori_loop(0, B, scl, 0)
                pltpu.sync_copy(buf, shout.at[dstb.at[p]], add=True)

                @pl.when(j + 2 < NB)
                def _():
                    idx_start(j + 2, p)
            return 0

        lax.fori_loop(0, NB // 2, body, 0)

        plsc.subcore_barrier()
        pltpu.sync_copy(shout.at[sl_nodes, :],
                        out_hbm.at[c, sl_nodes, :])

    return pass2


_pass1 = _make_pass1()
_pass2_l1 = _make_pass2(D_HID, BLK1, NBLK1)
_pass2_l2 = _make_pass2(D_IN, BLK1, NBLK1)


def kernel(x, edge_index, trainflag, W1, a_src1, a_dst1, b1, W2, a_src2, a_dst2, b2):
    ei = edge_index.astype(jnp.int32)
    loop = jnp.arange(N, dtype=jnp.int32)
    n_edges = ei.shape[1]
    n_pad = E_PAD - (n_edges + N)
    src = jnp.concatenate([ei[0], loop, jnp.zeros((n_pad,), jnp.int32)])
    dst = jnp.concatenate([ei[1], loop, jnp.full((n_pad,), N, jnp.int32)])
    src1 = src.reshape(NW, NBLK1, BLK1)
    dst1 = dst.reshape(NW, NBLK1, BLK1)
    xp = jnp.pad(x, ((0, NP - N), (0, 0)))

    h1, al1, am1 = _tc_layer1(xp, W1, a_src1, a_dst1)
    den1, ex1 = _pass1(src1, dst1, al1[0], al1[1], am1)
    p1 = _pass2_l1(src1, dst1, ex1, h1)

    midh, h2, al2, am2 = _tc_layer2(p1, den1, b1, W2, a_src2, a_dst2)
    den2, ex2 = _pass1(src1, dst1, al2[0], al2[1], am2)
    p2 = _pass2_l2(src1, dst1, ex2, h2)

    dec = _tc_final(p2, den2, b2)
    return (midh[:N], dec[:N])


# speedup vs baseline: 26.0310x; 1.0323x over previous
import functools

import jax
import jax.numpy as jnp
from jax import lax
from jax.experimental import pallas as pl
from jax.experimental.pallas import tpu as pltpu
from jax.experimental.pallas import tpu_sc as plsc

N = 10000
NP = 10240
NW = 32
BLK1 = 128
NBLK1 = 82
CHUNK = NBLK1 * BLK1
E_PAD = NW * CHUNK
BLK2 = 64
NBLK2 = CHUNK // BLK2
SLICE = NP // 16
D_IN = 128
D_HID = 64

_SC_PARAMS = pltpu.CompilerParams(
    needs_layout_passes=False, use_tc_tiling_on_sc=False)


def _alpha_outs(h, avs, avd):
    a_s = h @ avs
    a_d = h @ avd
    al = jnp.concatenate([a_s[None, :], a_d[None, :]], axis=0)
    am = jnp.concatenate(
        [jnp.full((1, 128), jnp.max(a_s), jnp.float32),
         jnp.full((1, 128), jnp.max(a_d), jnp.float32)], axis=0)
    return al, am


def _mm1_body(x_ref, w_ref, avs_ref, avd_ref, h_ref, al_ref, am_ref):
    h = jnp.dot(x_ref[...], w_ref[...], preferred_element_type=jnp.float32)
    h_ref[...] = h
    al, am = _alpha_outs(h, avs_ref[0, :], avd_ref[0, :])
    al_ref[...] = al
    am_ref[...] = am


def _tc_layer1(xp, W1, a_src1, a_dst1):
    return pl.pallas_call(
        _mm1_body,
        out_shape=(
            jax.ShapeDtypeStruct((NP, D_HID), jnp.float32),
            jax.ShapeDtypeStruct((2, NP), jnp.float32),
            jax.ShapeDtypeStruct((2, 128), jnp.float32),
        ),
    )(xp, W1, a_src1.reshape(1, -1), a_dst1.reshape(1, -1))


def _mm2_body(p_ref, den_ref, b_ref, w_ref, avs_ref, avd_ref,
              midh_ref, h2_ref, al_ref, am_ref):
    dsum = den_ref[0] + den_ref[1] + 1e-16
    m = (p_ref[0] + p_ref[1]) / dsum[:, None] + b_ref[0, :]
    m = jnp.maximum(m, 0.01 * m)
    rows = lax.broadcasted_iota(jnp.int32, (NP, D_HID), 0)
    m = jnp.where(rows < N, m, 0.0)
    midh_ref[...] = m
    h2 = jnp.dot(m, w_ref[...], preferred_element_type=jnp.float32)
    h2_ref[...] = h2
    al, am = _alpha_outs(h2, avs_ref[0, :], avd_ref[0, :])
    al_ref[...] = al
    am_ref[...] = am


def _tc_layer2(p, den, b1, W2, a_src2, a_dst2):
    return pl.pallas_call(
        _mm2_body,
        out_shape=(
            jax.ShapeDtypeStruct((NP, D_HID), jnp.float32),
            jax.ShapeDtypeStruct((NP, D_IN), jnp.float32),
            jax.ShapeDtypeStruct((2, NP), jnp.float32),
            jax.ShapeDtypeStruct((2, 128), jnp.float32),
        ),
    )(p, den, b1.reshape(1, -1), W2, a_src2.reshape(1, -1),
      a_dst2.reshape(1, -1))


def _fin_body(p_ref, den_ref, b_ref, dec_ref):
    dsum = den_ref[0] + den_ref[1] + 1e-16
    d = (p_ref[0] + p_ref[1]) / dsum[:, None] + b_ref[0, :]
    dec_ref[...] = jnp.maximum(d, 0.01 * d)


def _tc_final(p, den, b2):
    return pl.pallas_call(
        _fin_body,
        out_shape=jax.ShapeDtypeStruct((NP, D_IN), jnp.float32),
    )(p, den, b2.reshape(1, -1))


def _make_pass1():
    mesh = plsc.VectorSubcoreMesh(core_axis_name="c", subcore_axis_name="s")

    @functools.partial(
        pl.kernel,
        out_type=(
            jax.ShapeDtypeStruct((2, NP), jnp.float32),
            jax.ShapeDtypeStruct((NW, NBLK1, BLK1), jnp.float32),
        ),
        mesh=mesh,
        compiler_params=_SC_PARAMS,
        scratch_types=[
            pltpu.VMEM((NBLK1, BLK1), jnp.int32),
            pltpu.VMEM((NBLK1, BLK1), jnp.int32),
            pltpu.VMEM((NP,), jnp.float32),
            pltpu.VMEM((NP,), jnp.float32),
            pltpu.VMEM((NBLK1, BLK1), jnp.float32),
            pltpu.VMEM((SLICE,), jnp.float32),
            pltpu.VMEM((2, 128), jnp.float32),
            pltpu.VMEM_SHARED((NP,), jnp.float32),
        ],
    )
    def pass1(src_hbm, dst_hbm, as_hbm, ad_hbm, am_hbm, den_hbm, ex_hbm,
              srcv, dstv, asv, adv, exv, zbuf, amv, shden):
        c = lax.axis_index("c")
        s = lax.axis_index("s")
        wid = s * 2 + c
        pltpu.sync_copy(src_hbm.at[wid], srcv)
        pltpu.sync_copy(dst_hbm.at[wid], dstv)
        pltpu.sync_copy(as_hbm, asv)
        pltpu.sync_copy(ad_hbm, adv)
        pltpu.sync_copy(am_hbm, amv)

        zero16 = jnp.zeros((16,), jnp.float32)

        def zb(i, _):
            zbuf[pl.ds(i * 16, 16)] = zero16
            return 0

        lax.fori_loop(0, SLICE // 16, zb, 0)
        pltpu.sync_copy(zbuf, shden.at[pl.ds(s * SLICE, SLICE)])

        z = amv[0, pl.ds(0, 16)] + amv[1, pl.ds(0, 16)]
        M = jnp.maximum(z, 0.2 * z)

        plsc.subcore_barrier()

        def blk(j, _):
            for l in range(BLK1 // 16):
                sl = pl.ds(l * 16, 16)
                sidx = srcv[j, sl]
                didx = dstv[j, sl]
                a = plsc.load_gather(asv, [sidx])
                b = plsc.load_gather(adv, [didx])
                e = a + b
                e = jnp.maximum(e, 0.2 * e)
                exv[j, sl] = jnp.exp(e - M)
            pltpu.sync_copy(exv.at[j], shden.at[dstv.at[j]], add=True)
            return 0

        lax.fori_loop(0, NBLK1, blk, 0)

        pltpu.sync_copy(exv, ex_hbm.at[wid])
        plsc.subcore_barrier()
        pltpu.sync_copy(shden.at[pl.ds(s * SLICE, SLICE)],
                        den_hbm.at[c, pl.ds(s * SLICE, SLICE)])

    return pass1


def _make_pass2(D, B, NB):
    mesh = plsc.VectorSubcoreMesh(core_axis_name="c", subcore_axis_name="s")

    @functools.partial(
        pl.kernel,
        out_type=jax.ShapeDtypeStruct((2, NP, D), jnp.float32),
        mesh=mesh,
        compiler_params=_SC_PARAMS,
        scratch_types=[
            pltpu.VMEM((2, B), jnp.int32),
            pltpu.VMEM((2, B), jnp.int32),
            pltpu.VMEM((2, B), jnp.int32),
            pltpu.VMEM((2, B), jnp.float32),
            pltpu.VMEM((B, D), jnp.float32),
            pltpu.VMEM((B, D), jnp.float32),
            pltpu.VMEM_SHARED((NP, D), jnp.float32),
            pltpu.SemaphoreType.DMA,
            pltpu.SemaphoreType.DMA,
            pltpu.SemaphoreType.DMA,
        ],
    )
    def pass2(src_hbm, dst_hbm, ex_hbm, h_hbm, out_hbm,
              srcb, dstb, sdstb, exb, rows0, rows1, shout, semi, semg, sems):
        c = lax.axis_index("c")
        s = lax.axis_index("s")
        wid = s * 2 + c
        sl_nodes = pl.ds(s * SLICE, SLICE)

        zero16 = jnp.zeros((16,), jnp.float32)

        def zr(r, _):
            for k in range(D // 16):
                rows0[r, pl.ds(k * 16, 16)] = zero16
            return 0

        lax.fori_loop(0, B, zr, 0)
        for q in range(SLICE // B):
            pltpu.sync_copy(rows0, shout.at[pl.ds(s * SLICE + q * B, B), :])
        plsc.subcore_barrier()

        bufs = (rows0, rows1)

        def idx_start(j, p):
            pltpu.async_copy(src_hbm.at[wid, j], srcb.at[p], semi)
            pltpu.async_copy(dst_hbm.at[wid, j], dstb.at[p], semi)
            pltpu.async_copy(ex_hbm.at[wid, j], exb.at[p], semi)

        def idx_wait(p):
            pltpu.make_async_copy(src_hbm.at[wid, 0], srcb.at[p], semi).wait()
            pltpu.make_async_copy(dst_hbm.at[wid, 0], dstb.at[p], semi).wait()
            pltpu.make_async_copy(ex_hbm.at[wid, 0], exb.at[p], semi).wait()

        def gath_start(p):
            pltpu.async_copy(h_hbm.at[srcb.at[p]], bufs[p], semg)

        def gath_wait(p):
            pltpu.make_async_copy(h_hbm.at[srcb.at[p]], bufs[p], semg).wait()

        def sct_start(p):
            pltpu.async_copy(bufs[p], shout.at[sdstb.at[p]], sems, add=True)

        def sct_wait(p):
            pltpu.make_async_copy(h_hbm.at[pl.ds(0, B), :], bufs[p],
                                  sems).wait()

        idx_start(0, 0)
        idx_wait(0)
        gath_start(0)
        idx_start(1, 1)

        def body(jj, _):
            for p in range(2):
                j = jj * 2 + p
                buf = bufs[p]
                gath_wait(p)

                for l in range(B // 16):
                    sl = pl.ds(l * 16, 16)
                    sdstb[p, sl] = dstb[p, sl]

                @pl.when(j >= 1)
                def _():
                    sct_wait(1 - p)

                @pl.when(j + 1 < NB)
                def _():
                    idx_wait(1 - p)
                    gath_start(1 - p)

                pidx = jnp.full((16,), p, jnp.int32)

                def scl(r, _):
                    bidx = jnp.broadcast_to(r, (16,)).astype(jnp.int32)
                    sc = plsc.load_gather(exb, [pidx, bidx])
                    for k in range(D // 16):
                        sl = pl.ds(k * 16, 16)
                        buf[r, sl] = buf[r, sl] * sc
                    return 0

                lax.fori_loop(0, B, scl, 0)

                @pl.when(j + 2 < NB)
                def _():
                    idx_start(j + 2, p)

                sct_start(p)
            return 0

        lax.fori_loop(0, NB // 2, body, 0)

        sct_wait(1)
        plsc.subcore_barrier()
        pltpu.sync_copy(shout.at[sl_nodes, :],
                        out_hbm.at[c, sl_nodes, :])

    return pass2


_pass1 = _make_pass1()
_pass2_l1 = _make_pass2(D_HID, BLK1, NBLK1)
_pass2_l2 = _make_pass2(D_IN, BLK1, NBLK1)


def kernel(x, edge_index, trainflag, W1, a_src1, a_dst1, b1, W2, a_src2, a_dst2, b2):
    ei = edge_index.astype(jnp.int32)
    loop = jnp.arange(N, dtype=jnp.int32)
    n_edges = ei.shape[1]
    n_pad = E_PAD - (n_edges + N)
    src = jnp.concatenate([ei[0], loop, jnp.zeros((n_pad,), jnp.int32)])
    dst = jnp.concatenate([ei[1], loop, jnp.full((n_pad,), N, jnp.int32)])
    src1 = src.reshape(NW, NBLK1, BLK1)
    dst1 = dst.reshape(NW, NBLK1, BLK1)
    xp = jnp.pad(x, ((0, NP - N), (0, 0)))

    h1, al1, am1 = _tc_layer1(xp, W1, a_src1, a_dst1)
    den1, ex1 = _pass1(src1, dst1, al1[0], al1[1], am1)
    p1 = _pass2_l1(src1, dst1, ex1, h1)

    midh, h2, al2, am2 = _tc_layer2(p1, den1, b1, W2, a_src2, a_dst2)
    den2, ex2 = _pass1(src1, dst1, al2[0], al2[1], am2)
    p2 = _pass2_l2(src1, dst1, ex2, h2)

    dec = _tc_final(p2, den2, b2)
    return (midh[:N], dec[:N])
